# baseline (speedup 1.0000x reference)
import functools

import jax
import jax.numpy as jnp
from jax import lax
from jax.experimental import pallas as pl
from jax.experimental.pallas import tpu as pltpu
from jax.experimental.pallas import tpu_sc as plsc

_BS = 8
_NB = 64
_LANES = 16
_NTILES = 32
_CHUNKS_PER_TILE = 3
_NCHUNKS = _NTILES * _CHUNKS_PER_TILE
_PIECE = 32768


def _tc_body(x_ref, bd_ref, mags_ref, mx_out_ref, mx_ref):
    i = pl.program_id(0)
    n = pl.num_programs(0)
    t = x_ref[0, 0]
    bd = bd_ref[...]
    y = jnp.dot(t, bd, preferred_element_type=jnp.float32)
    d = jnp.dot(bd, y, preferred_element_type=jnp.float32)
    mag = jnp.abs(d)
    row = lax.broadcasted_iota(jnp.int32, mag.shape, 0)
    col = lax.broadcasted_iota(jnp.int32, mag.shape, 1)
    isdc = ((row & (_BS - 1)) == 0) & ((col & (_BS - 1)) == 0)
    mclean = jnp.where(isdc, 0.0, mag)
    mags_ref[0] = mclean

    @pl.when(i == 0)
    def _():
        mx_ref[0] = 0.0

    mx_ref[0] = jnp.maximum(mx_ref[0], jnp.max(mclean))

    @pl.when(i == n - 1)
    def _():
        mx_out_ref[0] = mx_ref[0]


def _sc_hist_body(mags_hbm, maxv_hbm, out_hbm, buf0, buf1, maxbuf, hist,
                  stage, sem0, sem1):
    nc = 2
    wid = lax.axis_index("s") * nc + lax.axis_index("c")
    lane = lax.iota(jnp.int32, _LANES)
    ones = jnp.ones((_LANES,), jnp.float32)
    zeros = jnp.zeros((_LANES,), jnp.float32)
    dccount = jnp.where(lane == 0, 2048.0, 0.0)

    pltpu.sync_copy(maxv_hbm, maxbuf)
    scale16 = float(_NB * _LANES) / (maxbuf[...] * 1.1)

    chunk_elems = mags_hbm.shape[0] // _NCHUNKS
    npieces = chunk_elems // _PIECE
    nq = _CHUNKS_PER_TILE * npieces
    bufs, sems = (buf0, buf1), (sem0, sem1)

    def _start(q):
        chunk = wid * _CHUNKS_PER_TILE + q // npieces
        off = chunk * chunk_elems + (q % npieces) * _PIECE
        return pltpu.async_copy(
            mags_hbm.at[pl.ds(off, _PIECE)], bufs[q % 2], sems[q % 2]
        )

    handles = {0: _start(0)}
    for q in range(nq):
        if q + 1 < nq:
            handles[q + 1] = _start(q + 1)

        if q % npieces == 0:
            def _zero(b, carry):
                hist[pl.ds(b * _LANES, _LANES)] = zeros
                return carry
            lax.fori_loop(0, _NB, _zero, 0)

        handles.pop(q).wait()
        buf = bufs[q % 2]

        @plsc.parallel_loop(0, _PIECE // _LANES, unroll=16)
        def _vec(k):
            v = buf[pl.ds(k * _LANES, _LANES)]
            idx = ((v * scale16).astype(jnp.int32) & ~(_LANES - 1)) + lane
            plsc.addupdate_scatter(hist, [idx], ones)

        if q % npieces == npieces - 1:
            chunk = wid * _CHUNKS_PER_TILE + q // npieces
            for g in range(_NB // _LANES):
                gs = [plsc.load_gather(
                          hist, [lane * _LANES + (g * _LANES * _LANES + l)])
                      for l in range(_LANES)]
                while len(gs) > 1:
                    gs = [a + b for a, b in zip(gs[::2], gs[1::2])]
                if g == 0:
                    gs[0] = gs[0] - dccount
                stage[pl.ds(g * _LANES, _LANES)] = gs[0]
            pltpu.sync_copy(stage, out_hbm.at[pl.ds(chunk * _NB, _NB)])


def kernel(x, dct_basis):
    B, C, H, W = x.shape
    bc = B * C
    bd = jnp.kron(jnp.eye(H // _BS, dtype=jnp.float32), dct_basis)

    mags, mx = pl.pallas_call(
        _tc_body,
        grid=(bc,),
        in_specs=[
            pl.BlockSpec((1, 1, H, W), lambda i: (i // C, i % C, 0, 0)),
            pl.BlockSpec((H, W), lambda i: (0, 0)),
        ],
        out_specs=[
            pl.BlockSpec((1, H, W), lambda i: (i, 0, 0)),
            pl.BlockSpec(memory_space=pltpu.SMEM),
        ],
        out_shape=[
            jax.ShapeDtypeStruct((bc, H, W), jnp.float32),
            jax.ShapeDtypeStruct((1,), jnp.float32),
        ],
        scratch_shapes=[pltpu.SMEM((1,), jnp.float32)],
    )(x, bd)

    maxv16 = jnp.broadcast_to(mx, (_LANES,))
    mags_flat = mags.reshape(-1)

    mesh = plsc.VectorSubcoreMesh(core_axis_name="c", subcore_axis_name="s")
    partials = pl.kernel(
        _sc_hist_body,
        out_type=jax.ShapeDtypeStruct((_NCHUNKS * _NB,), jnp.float32),
        mesh=mesh,
        compiler_params=pltpu.CompilerParams(needs_layout_passes=False),
        scratch_types=[
            pltpu.VMEM((_PIECE,), jnp.float32),
            pltpu.VMEM((_PIECE,), jnp.float32),
            pltpu.VMEM((_LANES,), jnp.float32),
            pltpu.VMEM((_NB * _LANES,), jnp.float32),
            pltpu.VMEM((_NB,), jnp.float32),
            pltpu.SemaphoreType.DMA,
            pltpu.SemaphoreType.DMA,
        ],
    )(mags_flat, maxv16)

    hist = partials.reshape(bc, 2, _NB).sum(axis=1) / (H * W)
    return hist.reshape(B, C * _NB)


# speedup vs baseline: 1.2460x; 1.2460x over previous
import functools

import jax
import jax.numpy as jnp
from jax import lax
from jax.experimental import pallas as pl
from jax.experimental.pallas import tpu as pltpu
from jax.experimental.pallas import tpu_sc as plsc

_BS = 8
_NB = 64
_LANES = 16
_NTILES = 32
_CHUNKS_PER_TILE = 3
_NCHUNKS = _NTILES * _CHUNKS_PER_TILE
_PIECE = 32768


def _tc_body(x_ref, bd_ref, mags_ref, mx_out_ref, mx_ref):
    i = pl.program_id(0)
    n = pl.num_programs(0)
    t = x_ref[0, 0]
    bd = bd_ref[...]
    y = jnp.dot(t, bd, preferred_element_type=jnp.float32)
    d = jnp.dot(bd, y, preferred_element_type=jnp.float32)
    mag = jnp.abs(d)
    row = lax.broadcasted_iota(jnp.int32, mag.shape, 0)
    col = lax.broadcasted_iota(jnp.int32, mag.shape, 1)
    isdc = ((row & (_BS - 1)) == 0) & ((col & (_BS - 1)) == 0)
    mclean = jnp.where(isdc, 0.0, mag)
    mags_ref[...] = mclean.reshape(mags_ref.shape)

    @pl.when(i == 0)
    def _():
        mx_ref[0] = 0.0

    mx_ref[0] = jnp.maximum(mx_ref[0], jnp.max(mclean))

    @pl.when(i == n - 1)
    def _():
        mx_out_ref[0] = mx_ref[0]


def _sc_hist_body(mags_hbm, maxv_hbm, out_hbm, buf0, buf1, maxbuf, hist,
                  stage, sem0, sem1):
    nc = 2
    wid = lax.axis_index("s") * nc + lax.axis_index("c")
    lane = lax.iota(jnp.int32, _LANES)
    ones = jnp.ones((_LANES,), jnp.float32)
    zeros = jnp.zeros((_LANES,), jnp.float32)
    dccount = jnp.where(lane == 0, 2048.0, 0.0)

    pltpu.sync_copy(maxv_hbm, maxbuf)
    scale16 = float(_NB * _LANES) / (maxbuf[...] * 1.1)

    chunk_elems = mags_hbm.shape[0] // _NCHUNKS
    npieces = chunk_elems // _PIECE
    nq = _CHUNKS_PER_TILE * npieces
    bufs, sems = (buf0, buf1), (sem0, sem1)

    def _start(q):
        chunk = wid * _CHUNKS_PER_TILE + q // npieces
        off = chunk * chunk_elems + (q % npieces) * _PIECE
        return pltpu.async_copy(
            mags_hbm.at[pl.ds(off, _PIECE)], bufs[q % 2], sems[q % 2]
        )

    handles = {0: _start(0)}
    for q in range(nq):
        if q + 1 < nq:
            handles[q + 1] = _start(q + 1)

        if q % npieces == 0:
            def _zero(b, carry):
                hist[pl.ds(b * _LANES, _LANES)] = zeros
                return carry
            lax.fori_loop(0, _NB, _zero, 0)

        handles.pop(q).wait()
        buf = bufs[q % 2]

        @plsc.parallel_loop(0, _PIECE // _LANES, unroll=8)
        def _vec(k):
            v = buf[pl.ds(k * _LANES, _LANES)]
            idx = ((v * scale16).astype(jnp.int32) & ~(_LANES - 1)) + lane
            plsc.addupdate_scatter(hist, [idx], ones)

        if q % npieces == npieces - 1:
            chunk = wid * _CHUNKS_PER_TILE + q // npieces
            for g in range(_NB // _LANES):
                gs = [plsc.load_gather(
                          hist, [lane * _LANES + (g * _LANES * _LANES + l)])
                      for l in range(_LANES)]
                while len(gs) > 1:
                    gs = [a + b for a, b in zip(gs[::2], gs[1::2])]
                if g == 0:
                    gs[0] = gs[0] - dccount
                stage[pl.ds(g * _LANES, _LANES)] = gs[0]
            pltpu.sync_copy(stage, out_hbm.at[pl.ds(chunk * _NB, _NB)])


def kernel(x, dct_basis):
    B, C, H, W = x.shape
    bc = B * C
    bd = jnp.kron(jnp.eye(H // _BS, dtype=jnp.float32), dct_basis)

    mags, mx = pl.pallas_call(
        _tc_body,
        grid=(bc,),
        in_specs=[
            pl.BlockSpec((1, 1, H, W), lambda i: (i // C, i % C, 0, 0)),
            pl.BlockSpec((H, W), lambda i: (0, 0)),
        ],
        out_specs=[
            pl.BlockSpec((H * W,), lambda i: (i,)),
            pl.BlockSpec(memory_space=pltpu.SMEM),
        ],
        out_shape=[
            jax.ShapeDtypeStruct((bc * H * W,), jnp.float32),
            jax.ShapeDtypeStruct((1,), jnp.float32),
        ],
        scratch_shapes=[pltpu.SMEM((1,), jnp.float32)],
    )(x, bd)

    maxv16 = jnp.broadcast_to(mx, (_LANES,))
    mags_flat = mags

    mesh = plsc.VectorSubcoreMesh(core_axis_name="c", subcore_axis_name="s")
    partials = pl.kernel(
        _sc_hist_body,
        out_type=jax.ShapeDtypeStruct((_NCHUNKS * _NB,), jnp.float32),
        mesh=mesh,
        compiler_params=pltpu.CompilerParams(needs_layout_passes=False),
        scratch_types=[
            pltpu.VMEM((_PIECE,), jnp.float32),
            pltpu.VMEM((_PIECE,), jnp.float32),
            pltpu.VMEM((_LANES,), jnp.float32),
            pltpu.VMEM((_NB * _LANES,), jnp.float32),
            pltpu.VMEM((_NB,), jnp.float32),
            pltpu.SemaphoreType.DMA,
            pltpu.SemaphoreType.DMA,
        ],
    )(mags_flat, maxv16)

    hist = partials.reshape(bc, 2, _NB).sum(axis=1) / (H * W)
    return hist.reshape(B, C * _NB)
